# baseline (speedup 1.0000x reference)
import jax
import jax.numpy as jnp
from jax.experimental import pallas as pl


_BM = 1024
_S = 4


def _matmul_block(*refs):
    z_refs = refs[:_S]
    wh_ref, wl_ref = refs[_S], refs[_S + 1]
    o_ref = refs[_S + 2]
    wh = wh_ref[...]
    wl = wl_ref[...]
    for j in range(_S):
        z = z_refs[j][...]
        zh = z.astype(jnp.bfloat16)
        zl = (z - zh.astype(jnp.float32)).astype(jnp.bfloat16)
        acc = jnp.dot(zh, wh, preferred_element_type=jnp.float32)
        acc += jnp.dot(zl, wh, preferred_element_type=jnp.float32)
        acc += jnp.dot(zh, wl, preferred_element_type=jnp.float32)
        o_ref[pl.ds(j * _BM, _BM), :] = acc


def kernel(Z, W):
    M, K = Z.shape
    N = W.shape[1]
    Wh = W.astype(jnp.bfloat16)
    Wl = (W - Wh.astype(jnp.float32)).astype(jnp.bfloat16)

    def _z_spec(j):
        return pl.BlockSpec((_BM, K), lambda i, j=j: (_S * i + j, 0))

    return pl.pallas_call(
        _matmul_block,
        grid=(M // (_S * _BM),),
        in_specs=[_z_spec(j) for j in range(_S)] + [
            pl.BlockSpec((K, N), lambda i: (0, 0)),
            pl.BlockSpec((K, N), lambda i: (0, 0)),
        ],
        out_specs=pl.BlockSpec((_S * _BM, N), lambda i: (i, 0)),
        out_shape=jax.ShapeDtypeStruct((M, N), jnp.float32),
    )(*([Z] * _S), Wh, Wl)
